# Initial kernel scaffold; baseline (speedup 1.0000x reference)
import jax
import jax.numpy as jnp
from jax.experimental import pallas as pl


def kernel(full_rec_data, res_index, n_feat, W1, b1, gamma, beta, Wa, ba):
    raise NotImplementedError("write your pallas kernel here")


# speedup vs baseline: 6.4696x; 6.4696x over previous
import jax
import jax.numpy as jnp
from jax import lax
from jax.experimental import pallas as pl
from jax.experimental.pallas import tpu as pltpu

RA_ = 10000

B1 = 8000
B2 = 1280
K = 128
ACC_ROWS = RA_ + K + 16
PW = 8


def _lrelu(x):
    return jnp.where(x >= 0, x, 0.01 * x)


def _pass1_body(x_ref, wa_ref, max_ref):
    i = pl.program_id(0)

    @pl.when(i == 0)
    def _():
        max_ref[...] = jnp.full_like(max_ref, -jnp.inf)

    act = _lrelu(x_ref[...])
    t = jnp.sum(act * wa_ref[...], axis=-1)
    m = jnp.max(t)
    max_ref[...] = jnp.maximum(max_ref[...], m)


def _pass2_body(x_ref, seg_ref, max_ref, w1_ref, b1_ref, g_ref, be_ref,
                wa_ref, out_ref, acc_ref, accp_ref):
    i = pl.program_id(0)
    nb = pl.num_programs(0)

    @pl.when(i == 0)
    def _():
        acc_ref[...] = jnp.zeros_like(acc_ref)
        accp_ref[...] = jnp.zeros_like(accp_ref)

    x = x_ref[...]
    act = _lrelu(x)
    hid = lax.dot_general(act, w1_ref[...], (((1,), (0,)), ((), ())),
                          preferred_element_type=jnp.float32)
    hid = hid + b1_ref[...]
    mu = jnp.mean(hid, axis=-1, keepdims=True)
    var = jnp.mean((hid - mu) ** 2, axis=-1, keepdims=True)
    feat = (hid - mu) * lax.rsqrt(var + 1e-5) * g_ref[...] + be_ref[...]

    t = jnp.sum(act * wa_ref[...], axis=-1, keepdims=True)
    m = max_ref[0, 0]
    p = jnp.exp(t - m)
    g = feat * p
    p8 = jnp.broadcast_to(p, (B2, PW))

    seg = seg_ref[...]
    s_first = seg_ref[0, 0]
    s_last = seg_ref[B2 - 1, 0]
    base0 = (s_first // 8) * 8
    nch = (s_last - base0) // K + 1

    col = lax.broadcasted_iota(jnp.int32, (1, K), 1)

    def chunk(c, _):
        start = base0 + c * K
        oh = (seg == (start + col)).astype(jnp.float32)
        sums = lax.dot_general(oh, g, (((0,), (0,)), ((), ())),
                               preferred_element_type=jnp.float32)
        psums = lax.dot_general(oh, p8, (((0,), (0,)), ((), ())),
                                preferred_element_type=jnp.float32)
        acc_ref[pl.ds(start, K), :] += sums
        accp_ref[pl.ds(start, K), :] += psums
        return 0

    lax.fori_loop(0, nch, chunk, 0)

    @pl.when(i == nb - 1)
    def _():
        denom = accp_ref[:RA_, 0:1]
        denom = jnp.where(denom == 0.0, 1.0, denom)
        out_ref[...] = acc_ref[:RA_, :] / denom


def kernel(full_rec_data, res_index, n_feat, W1, b1, gamma, beta, Wa, ba):
    x = full_rec_data
    rf, f1 = x.shape
    nf = W1.shape[1]
    nb1 = rf // B1
    nb2 = rf // B2
    wa_row = Wa.reshape(1, f1)
    seg_col = res_index.reshape(rf, 1)

    gmax = pl.pallas_call(
        _pass1_body,
        grid=(nb1,),
        in_specs=[
            pl.BlockSpec((B1, f1), lambda i: (i, 0)),
            pl.BlockSpec((1, f1), lambda i: (0, 0)),
        ],
        out_specs=pl.BlockSpec((8, 128), lambda i: (0, 0)),
        out_shape=jax.ShapeDtypeStruct((8, 128), jnp.float32),
    )(x, wa_row)

    out = pl.pallas_call(
        _pass2_body,
        grid=(nb2,),
        in_specs=[
            pl.BlockSpec((B2, f1), lambda i: (i, 0)),
            pl.BlockSpec((B2, 1), lambda i: (i, 0)),
            pl.BlockSpec((8, 128), lambda i: (0, 0)),
            pl.BlockSpec((f1, nf), lambda i: (0, 0)),
            pl.BlockSpec((1, nf), lambda i: (0, 0)),
            pl.BlockSpec((1, nf), lambda i: (0, 0)),
            pl.BlockSpec((1, nf), lambda i: (0, 0)),
            pl.BlockSpec((1, f1), lambda i: (0, 0)),
        ],
        out_specs=pl.BlockSpec((RA_, nf), lambda i: (0, 0)),
        out_shape=jax.ShapeDtypeStruct((RA_, nf), jnp.float32),
        scratch_shapes=[
            pltpu.VMEM((ACC_ROWS, nf), jnp.float32),
            pltpu.VMEM((ACC_ROWS, PW), jnp.float32),
        ],
    )(x, seg_col, gmax, W1, b1.reshape(1, nf), gamma.reshape(1, nf),
      beta.reshape(1, nf), wa_row)
    return out


# speedup vs baseline: 13.9738x; 2.1599x over previous
import jax
import jax.numpy as jnp
from jax import lax
from jax.experimental import pallas as pl
from jax.experimental.pallas import tpu as pltpu

RA_ = 10000

B2 = 2560
K = 128
ACC_ROWS = RA_ + K + 16
PW = 8


def _lrelu(x):
    return jnp.where(x >= 0, x, 0.01 * x)


def _body(x_ref, seg_ref, w1_ref, b1_ref, gb_ref, mean_ref, wao_ref,
          out_ref, acc_ref, accp_ref, m_ref):
    i = pl.program_id(0)
    nb = pl.num_programs(0)

    @pl.when(i == 0)
    def _():
        acc_ref[...] = jnp.zeros_like(acc_ref)
        accp_ref[...] = jnp.zeros_like(accp_ref)
        m_ref[0, 0] = -jnp.inf

    x = x_ref[...]
    act = _lrelu(x)
    hid = lax.dot_general(act, w1_ref[...], (((1,), (0,)), ((), ())),
                          preferred_element_type=jnp.float32)
    hid = hid + b1_ref[...]
    mu = lax.dot_general(hid, mean_ref[...], (((1,), (0,)), ((), ())),
                         preferred_element_type=jnp.float32)
    sq = hid * hid
    msq = lax.dot_general(sq, mean_ref[...], (((1,), (0,)), ((), ())),
                          preferred_element_type=jnp.float32)
    var = msq - mu * mu
    rs = lax.rsqrt(var + 1e-5)
    feat = (hid - mu) * rs * gb_ref[0:1, :] + gb_ref[1:2, :]

    t = lax.dot_general(act, wao_ref[...], (((1,), (0,)), ((), ())),
                        preferred_element_type=jnp.float32)
    bm = jnp.max(t)
    m_old = m_ref[0, 0]
    m_new = jnp.maximum(m_old, bm)
    m_ref[0, 0] = m_new

    p = jnp.exp(t - m_new)
    g = feat * p
    p8 = p[:, 0:PW]

    seg_row = seg_ref[0]
    s_first = seg_ref[0, 0, 0]
    s_last = seg_ref[0, 0, B2 - 1]
    base0 = (s_first // 8) * 8
    nch = (s_last - base0) // K + 1

    factor = jnp.exp(m_old - m_new)
    acc_ref[pl.ds(s_first, 1), :] *= factor
    accp_ref[pl.ds(s_first, 1), :] *= factor

    rowk = lax.broadcasted_iota(jnp.int32, (K, 1), 0)

    def chunk(c, _):
        start = base0 + c * K
        oht = (seg_row == (start + rowk)).astype(jnp.float32)
        sums = lax.dot_general(oht, g, (((1,), (0,)), ((), ())),
                               preferred_element_type=jnp.float32)
        psums = lax.dot_general(oht, p8, (((1,), (0,)), ((), ())),
                                preferred_element_type=jnp.float32)
        acc_ref[pl.ds(start, K), :] += sums
        accp_ref[pl.ds(start, K), :] += psums
        return 0

    lax.fori_loop(0, nch, chunk, 0)

    @pl.when(i == nb - 1)
    def _():
        denom = accp_ref[:RA_, 0:1]
        denom = jnp.where(denom == 0.0, 1.0, denom)
        out_ref[...] = acc_ref[:RA_, :] / denom


def kernel(full_rec_data, res_index, n_feat, W1, b1, gamma, beta, Wa, ba):
    x = full_rec_data
    rf, f1 = x.shape
    nf = W1.shape[1]
    nb2 = rf // B2
    seg3 = res_index.reshape(nb2, 1, B2)
    gb = jnp.stack([gamma, beta], axis=0)
    mean_mat = jnp.full((nf, nf), 1.0 / nf, dtype=jnp.float32)
    wa_outer = jnp.broadcast_to(Wa, (f1, nf)).astype(jnp.float32)

    out = pl.pallas_call(
        _body,
        grid=(nb2,),
        in_specs=[
            pl.BlockSpec((B2, f1), lambda i: (i, 0)),
            pl.BlockSpec((1, 1, B2), lambda i: (i, 0, 0)),
            pl.BlockSpec((f1, nf), lambda i: (0, 0)),
            pl.BlockSpec((1, nf), lambda i: (0, 0)),
            pl.BlockSpec((2, nf), lambda i: (0, 0)),
            pl.BlockSpec((nf, nf), lambda i: (0, 0)),
            pl.BlockSpec((f1, nf), lambda i: (0, 0)),
        ],
        out_specs=pl.BlockSpec((RA_, nf), lambda i: (0, 0)),
        out_shape=jax.ShapeDtypeStruct((RA_, nf), jnp.float32),
        scratch_shapes=[
            pltpu.VMEM((ACC_ROWS, nf), jnp.float32),
            pltpu.VMEM((ACC_ROWS, PW), jnp.float32),
            pltpu.SMEM((1, 1), jnp.float32),
        ],
    )(x, seg3, W1, b1.reshape(1, nf), gb, mean_mat, wa_outer)
    return out
